# baseline (speedup 1.0000x reference)
import functools

import jax
import jax.numpy as jnp
from jax import lax
from jax.experimental import pallas as pl
from jax.experimental.pallas import tpu as pltpu
from jax.experimental.pallas import tpu_sc as plsc

EMB_DIM = 64
BATCH = 16384
NUM_CORES = 2
NUM_SUBCORES = 16
NUM_WORKERS = NUM_CORES * NUM_SUBCORES
BPW = BATCH // NUM_WORKERS
LANES = 16
CHUNKS = EMB_DIM // LANES

_mesh = plsc.VectorSubcoreMesh(core_axis_name="c", subcore_axis_name="s")


@functools.partial(
    pl.kernel,
    mesh=_mesh,
    out_type=jax.ShapeDtypeStruct((BATCH, EMB_DIM), jnp.float32),
    compiler_params=pltpu.CompilerParams(use_tc_tiling_on_sc=False),
    scratch_types=[
        pltpu.VMEM((BPW,), jnp.int32),
        pltpu.VMEM((BPW,), jnp.int32),
        pltpu.VMEM((BPW,), jnp.int32),
        pltpu.VMEM((BPW, EMB_DIM), jnp.float32),
        pltpu.VMEM((BPW, EMB_DIM), jnp.float32),
        pltpu.VMEM((BPW, EMB_DIM), jnp.float32),
        pltpu.SemaphoreType.DMA,
    ],
)
def _transe_sc(ent_hbm, rel_hbm, head_hbm, ridx_hbm, tail_hbm, out_hbm,
               idx_h, idx_r, idx_t, hbuf, rbuf, tbuf, sem):
    wid = lax.axis_index("s") * NUM_CORES + lax.axis_index("c")
    base = wid * BPW

    pltpu.sync_copy(head_hbm.at[pl.ds(base, BPW)], idx_h)
    pltpu.sync_copy(ridx_hbm.at[pl.ds(base, BPW)], idx_r)
    pltpu.sync_copy(tail_hbm.at[pl.ds(base, BPW)], idx_t)

    ch = pltpu.async_copy(ent_hbm.at[idx_h], hbuf, sem)
    cr = pltpu.async_copy(rel_hbm.at[idx_r], rbuf, sem)
    ct = pltpu.async_copy(ent_hbm.at[idx_t], tbuf, sem)
    ch.wait()
    cr.wait()
    ct.wait()

    def body(i, carry):
        for j in range(CHUNKS):
            sl = pl.ds(j * LANES, LANES)
            hbuf[i, sl] = hbuf[i, sl] + rbuf[i, sl] - tbuf[i, sl]
        return carry

    lax.fori_loop(0, BPW, body, 0)

    pltpu.sync_copy(hbuf, out_hbm.at[pl.ds(base, BPW)])


def kernel(head, relation, tail, ent_emb, rel_emb):
    return _transe_sc(
        ent_emb,
        rel_emb,
        head.reshape(BATCH),
        relation.reshape(BATCH),
        tail.reshape(BATCH),
    )


# speedup vs baseline: 1.3647x; 1.3647x over previous
import functools

import jax
import jax.numpy as jnp
from jax import lax
from jax.experimental import pallas as pl
from jax.experimental.pallas import tpu as pltpu
from jax.experimental.pallas import tpu_sc as plsc

ENT_ROWS = 1000000
REL_ROWS = 1000
EMB_DIM = 64
BATCH = 16384
NUM_CORES = 2
NUM_SUBCORES = 16
NUM_WORKERS = NUM_CORES * NUM_SUBCORES
BPW = BATCH // NUM_WORKERS
CHUNK = 32
NCHUNKS = BPW // CHUNK

_mesh = plsc.VectorSubcoreMesh(core_axis_name="c", subcore_axis_name="s")


@functools.partial(
    pl.kernel,
    mesh=_mesh,
    out_type=jax.ShapeDtypeStruct((BATCH, EMB_DIM), jnp.float32),
    compiler_params=pltpu.CompilerParams(use_tc_tiling_on_sc=True,
                                         needs_layout_passes=False),
    scratch_types=[
        pltpu.VMEM((BPW,), jnp.int32),
        pltpu.VMEM((BPW,), jnp.int32),
        pltpu.VMEM((BPW,), jnp.int32),
        pltpu.VMEM((CHUNK,), jnp.int32),
        pltpu.VMEM((CHUNK,), jnp.int32),
        pltpu.VMEM((CHUNK,), jnp.int32),
        pltpu.VMEM((CHUNK * 8, EMB_DIM), jnp.float32),
        pltpu.VMEM((CHUNK * 8, EMB_DIM), jnp.float32),
        pltpu.VMEM((CHUNK * 8, EMB_DIM), jnp.float32),
        pltpu.VMEM((CHUNK, EMB_DIM), jnp.float32),
        pltpu.SemaphoreType.DMA,
    ],
)
def _transe_sc(ent_hbm, rel_hbm, head_hbm, ridx_hbm, tail_hbm, out_hbm,
               idxh, idxr, idxt, gih, git, gir, hdst, tdst, rdst, outb, sem):
    wid = lax.axis_index("s") * NUM_CORES + lax.axis_index("c")
    base = wid * BPW

    pltpu.sync_copy(head_hbm.at[pl.ds(base, BPW)], idxh)
    pltpu.sync_copy(ridx_hbm.at[pl.ds(base, BPW)], idxr)
    pltpu.sync_copy(tail_hbm.at[pl.ds(base, BPW)], idxt)

    iota = lax.iota(jnp.int32, 16)

    def do_chunk(chunk, carry):
        cbase = chunk * CHUNK

        subs = []
        copies = []
        for lg in range(CHUNK // 16):
            sl = pl.ds(cbase + lg * 16, 16)
            e_h = idxh[sl]
            e_t = idxt[sl]
            e_r = idxr[sl]
            th = (e_h >> 3) << 3
            tt = (e_t >> 3) << 3
            tr = (e_r >> 3) << 3
            for j in range(16):
                dsl = pl.ds((lg * 16 + j) * 8, 8)
                oh = pl.multiple_of(th[j], 8)
                ot = pl.multiple_of(tt[j], 8)
                orr = pl.multiple_of(tr[j], 8)
                copies.append(pltpu.async_copy(
                    ent_hbm.at[pl.ds(oh, 8), :], hdst.at[dsl, :], sem))
                copies.append(pltpu.async_copy(
                    ent_hbm.at[pl.ds(ot, 8), :], tdst.at[dsl, :], sem))
                copies.append(pltpu.async_copy(
                    rel_hbm.at[pl.ds(orr, 8), :], rdst.at[dsl, :], sem))
            subs.append((e_h & 7, e_t & 7, e_r & 7))
        for c in copies:
            c.wait()

        def extract(d, c):
            cols = d * 16 + iota
            for lg, (s_h, s_t, s_r) in enumerate(subs):
                for j in range(16):
                    l = lg * 16 + j
                    hv = plsc.load_gather(
                        hdst, [jnp.broadcast_to(l * 8 + s_h[j], (16,)), cols])
                    tv = plsc.load_gather(
                        tdst, [jnp.broadcast_to(l * 8 + s_t[j], (16,)), cols])
                    rv = plsc.load_gather(
                        rdst, [jnp.broadcast_to(l * 8 + s_r[j], (16,)), cols])
                    outb[l, pl.ds(d * 16, 16)] = hv + rv - tv
            return c

        lax.fori_loop(0, EMB_DIM // 16, extract, 0)

        pltpu.sync_copy(outb, out_hbm.at[pl.ds(base + cbase, CHUNK)])
        return carry

    lax.fori_loop(0, NCHUNKS, do_chunk, 0)


def kernel(head, relation, tail, ent_emb, rel_emb):
    return _transe_sc(
        ent_emb,
        rel_emb,
        head.reshape(BATCH),
        relation.reshape(BATCH),
        tail.reshape(BATCH),
    )
